# baseline (speedup 1.0000x reference)
import functools

import numpy as np

import jax
import jax.numpy as jnp
from jax import lax
from jax.experimental import pallas as pl
from jax.experimental.pallas import tpu as pltpu
from jax.experimental.pallas import tpu_sc as plsc

EPS_BN = 1e-5
BLK = 1000
NC, NS = 2, 16
NW = NC * NS
NBUF = 4


def _fold_pn(params):
    W1, b1, g1, be1, W2, b2, g2, be2 = params
    s = 1.0 / jnp.sqrt(1.0 + EPS_BN)
    A1 = W1.T * (g1 * s)[None, :]
    c1 = (b1 * (g1 * s) + be1).reshape(1, -1)
    A2 = W2.T * (g2 * s)[None, :]
    c2 = (b2 * (g2 * s) + be2).reshape(1, -1)
    return A1, c1, A2, c2


def _kron9(a):
    r, c = a.shape
    eye = jnp.asarray(np.eye(9, dtype=np.float32))
    return (eye[:, None, :, None] * a[None, :, None, :]).reshape(9 * r, 9 * c)


_P54 = np.zeros((216, 54), np.float32)
for _j in range(9):
    for _c in range(6):
        _P54[24 * _j + _c, 6 * _j + _c] = 1.0
_Q54 = np.tile(-np.eye(6, dtype=np.float32), (1, 9))


def _dot(x, y):
    return jnp.dot(x, y, preferred_element_type=jnp.float32)


def _pn2(x, a1, c1, a2, c2):
    return _dot(jnp.maximum(_dot(x, a1) + c1, 0.0), a2) + c2


def _colsumsq_kernel(x_ref, o_ref):
    p = jnp.sum(x_ref[...] * x_ref[...], axis=0, keepdims=True)

    @pl.when(pl.program_id(0) == 0)
    def _():
        o_ref[...] = p

    @pl.when(pl.program_id(0) > 0)
    def _():
        o_ref[...] += p


def _prep_kernel(ss_ref, x_ref, pn_ref, a1_ref, c1_ref, a2_ref, c2_ref,
                 t1_ref, nb_ref, pf6_ref):
    x = x_ref[...]
    inv = 1.0 / jnp.maximum(jnp.sqrt(ss_ref[0:1, 0:3]), 1e-12)
    pf6 = jnp.concatenate([x[:, 0:3] * inv, x[:, 3:6] * (1.0 / 255.0)], axis=1)
    f1 = _pn2(pf6, a1_ref[...], c1_ref[...], a2_ref[...], c2_ref[...])
    sel = jnp.concatenate([x[:, 0:3], x[:, 6:9]], axis=1)
    t1_ref[...] = jnp.concatenate(
        [sel, f1, jnp.zeros((BLK, 6), jnp.float32)], axis=1)
    pf6_ref[...] = pf6
    row = pl.program_id(0) * BLK + lax.broadcasted_iota(jnp.int32, (BLK, 9), 0)
    pnv = pn_ref[...]
    nb_ref[...] = jnp.where(pnv == 0, row, pnv)


def _make_gather(d, ch, b_pad):
    bpw = b_pad // NW
    n_ch = bpw // ch
    assert bpw % ch == 0 and ch % 8 == 0 and n_ch % NBUF == 0 and n_ch >= 2 * NBUF
    mesh = plsc.VectorSubcoreMesh(core_axis_name="c", subcore_axis_name="s")

    @functools.partial(
        pl.kernel,
        mesh=mesh,
        out_type=jax.ShapeDtypeStruct((b_pad, d), jnp.float32),
        scratch_types=[
            pltpu.VMEM((bpw,), jnp.int32),
            pltpu.VMEM((NBUF, ch, d), jnp.float32),
        ] + [pltpu.SemaphoreType.DMA] * (2 * NBUF),
        compiler_params=pltpu.CompilerParams(use_tc_tiling_on_sc=False),
    )
    def gather(table_hbm, idx_hbm, out_hbm, idx_v, bufs, *sems):
        wid = lax.axis_index("s") * NC + lax.axis_index("c")
        base = wid * bpw
        gsems, wsems = sems[:NBUF], sems[NBUF:]
        pltpu.sync_copy(idx_hbm.at[pl.ds(base, bpw)], idx_v)

        def g_copy(g, b):
            return pltpu.make_async_copy(
                table_hbm.at[idx_v.at[pl.ds(g * ch, ch)]], bufs.at[b], gsems[b])

        def w_copy(g, b):
            return pltpu.make_async_copy(
                bufs.at[b], out_hbm.at[pl.ds(base + g * ch, ch)], wsems[b])

        for b in range(NBUF):
            g_copy(b, b).start()

        def body(k, _):
            for b in range(NBUF):
                g = NBUF * k + b
                g_copy(g, b).wait()
                w_copy(g, b).start()
                w_copy(g, b).wait()
                g_copy(g + NBUF, b).start()
            return 0

        lax.fori_loop(0, n_ch // NBUF - 1, body, 0)
        for b in range(NBUF):
            g = n_ch - NBUF + b
            g_copy(g, b).wait()
            w_copy(g, b).start()
            w_copy(g, b).wait()

    return gather


def _s1_kernel(e_ref, t1_ref, a1bd, selb, c1t, a2emb, c2emb,
               u1p, uc1, u2a, uc2, fa1, fc1, fa2, fc2, p54, q54,
               feas_ref, t2_ref, xyz_ref):
    e = e_ref[...]
    sel6 = t1_ref[:, 0:6]
    h = jnp.maximum(_dot(e, a1bd[...]) + _dot(sel6, selb[...]) + c1t[...], 0.0)
    wemb = _dot(h, a2emb[...]) + c2emb[...]
    m = e * wemb
    feas = _pn2(m, u1p[...], uc1[...], u2a[...], uc2[...])
    feas_ref[...] = feas
    t2_ref[...] = _pn2(feas, fa1[...], fc1[...], fa2[...], fc2[...])
    xyz_ref[...] = _dot(e, p54[...]) + _dot(sel6, q54[...])


def _s23_kernel(e_ref, xyz_ref, a1bd, c1t, a2bd, c2t,
                u1a, uc1, u2a, uc2, *rest):
    has_next = len(rest) == 6
    e = e_ref[...]
    h = jnp.maximum(_dot(xyz_ref[...], a1bd[...]) + c1t[...], 0.0)
    w = _dot(h, a2bd[...]) + c2t[...]
    m = e * w
    feas = _pn2(m, u1a[...], uc1[...], u2a[...], uc2[...])
    if has_next:
        fa1, fc1, fa2, fc2, feas_ref, tn_ref = rest
        feas_ref[...] = feas
        tn_ref[...] = _pn2(feas, fa1[...], fc1[...], fa2[...], fc2[...])
    else:
        (feas_ref,) = rest
        feas_ref[...] = feas


def _full_spec(shape):
    return pl.BlockSpec(shape, lambda i: (0, 0))


def _row_spec(w):
    return pl.BlockSpec((BLK, w), lambda i: (i, 0))


Q = 4


def kernel(points_features, points_neighbor, p1f, p1w, p1u, p2f, p2w, p2u,
           p3f, p3w, p3u):
    n, m = points_features.shape[0], points_neighbor.shape[1]
    nq = n // Q
    bq = nq * m
    ch24, ch48 = 600, 360
    bq_pad = ((bq + 8 * NW - 1) // (8 * NW)) * (8 * NW)
    while ((bq_pad // NW) % (NBUF * ch24) or (bq_pad // NW) % (NBUF * ch48)
           or bq_pad % 9):
        bq_pad += 8 * NW
    grid = (n // BLK,)
    gridq = (nq // BLK,)
    qb = nq // BLK

    f1p = _fold_pn(p1f)
    w1a1, w1c1, w1a2, w1c2 = _fold_pn(p1w)
    u1a1, u1c1, u1a2, u1c2 = _fold_pn(p1u)
    f2p = _fold_pn(p2f)
    w2a1, w2c1, w2a2, w2c2 = _fold_pn(p2w)
    u2a1, u2c1, u2a2, u2c2 = _fold_pn(p2u)
    f3p = _fold_pn(p3f)
    w3a1, w3c1, w3a2, w3c2 = _fold_pn(p3w)
    u3a1, u3c1, u3a2, u3c2 = _fold_pn(p3u)

    z = jnp.zeros
    s1_a1bd = _kron9(jnp.concatenate([w1a1, z((18, 12), jnp.float32)], axis=0))
    s1_selb = jnp.tile(-w1a1, (1, 9))
    s1_c1t = jnp.tile(w1c1, (1, 9))
    pad6 = z((12, 6), jnp.float32)
    s1_a2emb = _kron9(jnp.concatenate([pad6, w1a2, pad6], axis=1))
    s1_c2emb = jnp.tile(jnp.concatenate(
        [z((1, 6), jnp.float32), w1c2, z((1, 6), jnp.float32)], axis=1), (1, 9))
    s1_u1p = jnp.concatenate([
        z((9, 6, 12), jnp.float32), u1a1.reshape(9, 12, 12),
        z((9, 6, 12), jnp.float32)], axis=1).reshape(216, 12)
    s1_p54 = jnp.asarray(_P54)
    s1_q54 = jnp.asarray(_Q54)

    s2_a1bd = _kron9(w2a1)
    s2_c1t = jnp.tile(w2c1, (1, 9))
    s2_a2bd = _kron9(w2a2)
    s2_c2t = jnp.tile(w2c2, (1, 9))
    s3_a1bd = _kron9(w3a1)
    s3_c1t = jnp.tile(w3c1, (1, 9))
    s3_a2bd = _kron9(w3a2)
    s3_c2t = jnp.tile(w3c2, (1, 9))

    ss = pl.pallas_call(
        _colsumsq_kernel,
        grid=grid,
        in_specs=[_row_spec(9)],
        out_specs=pl.BlockSpec((1, 9), lambda i: (0, 0)),
        out_shape=jax.ShapeDtypeStruct((1, 9), jnp.float32),
    )(points_features)

    t1, nb, pf6 = pl.pallas_call(
        _prep_kernel,
        grid=grid,
        in_specs=[_full_spec((1, 9)), _row_spec(9), _row_spec(9)]
        + [_full_spec(x.shape) for x in f1p],
        out_specs=[_row_spec(24), _row_spec(9), _row_spec(6)],
        out_shape=[jax.ShapeDtypeStruct((n, 24), jnp.float32),
                   jax.ShapeDtypeStruct((n, 9), jnp.int32),
                   jax.ShapeDtypeStruct((n, 6), jnp.float32)],
    )(ss, points_features, points_neighbor, *f1p)

    idxs = [jnp.pad(nb[q * nq:(q + 1) * nq].reshape(-1), (0, bq_pad - bq))
            for q in range(Q)]

    gather24 = _make_gather(24, ch24, bq_pad)
    gather48 = _make_gather(48, ch48, bq_pad)

    def qspec(w, q):
        return pl.BlockSpec((BLK, w), lambda i, _q=q: (i + _q * qb, 0))

    s1_w = (s1_a1bd, s1_selb, s1_c1t, s1_a2emb, s1_c2emb,
            s1_u1p, u1c1, u1a2, u1c2, *f2p, s1_p54, s1_q54)
    s2_w = (s2_a1bd, s2_c1t, s2_a2bd, s2_c2t, u2a1, u2c1, u2a2, u2c2, *f3p)
    s3_w = (s3_a1bd, s3_c1t, s3_a2bd, s3_c2t, u3a1, u3c1, u3a2, u3c2)

    feas1s, t2s, xyzs = [], [], []
    for q in range(Q):
        e1 = gather24(t1, idxs[q]).reshape(bq_pad // 9, 9 * 24)
        f_, t_, x_ = pl.pallas_call(
            _s1_kernel,
            grid=gridq,
            in_specs=[_row_spec(216), qspec(24, q)]
            + [_full_spec(x.shape) for x in s1_w],
            out_specs=[_row_spec(12), _row_spec(24), _row_spec(54)],
            out_shape=[jax.ShapeDtypeStruct((nq, 12), jnp.float32),
                       jax.ShapeDtypeStruct((nq, 24), jnp.float32),
                       jax.ShapeDtypeStruct((nq, 54), jnp.float32)],
        )(e1, t1, *s1_w)
        feas1s.append(f_), t2s.append(t_), xyzs.append(x_)
    t2 = jnp.concatenate(t2s, axis=0)

    feas2s, t3s = [], []
    for q in range(Q):
        e2 = gather24(t2, idxs[q]).reshape(bq_pad // 9, 9 * 24)
        f_, t_ = pl.pallas_call(
            _s23_kernel,
            grid=gridq,
            in_specs=[_row_spec(216), _row_spec(54)]
            + [_full_spec(x.shape) for x in s2_w],
            out_specs=[_row_spec(24), _row_spec(48)],
            out_shape=[jax.ShapeDtypeStruct((nq, 24), jnp.float32),
                       jax.ShapeDtypeStruct((nq, 48), jnp.float32)],
        )(e2, xyzs[q], *s2_w)
        feas2s.append(f_), t3s.append(t_)
    t3 = jnp.concatenate(t3s, axis=0)

    feas3s = []
    for q in range(Q):
        e3 = gather48(t3, idxs[q]).reshape(bq_pad // 9, 9 * 48)
        f_ = pl.pallas_call(
            _s23_kernel,
            grid=gridq,
            in_specs=[_row_spec(432), _row_spec(54)]
            + [_full_spec(x.shape) for x in s3_w],
            out_specs=_row_spec(48),
            out_shape=jax.ShapeDtypeStruct((nq, 48), jnp.float32),
        )(e3, xyzs[q], *s3_w)
        feas3s.append(f_)

    return jnp.concatenate(
        [jnp.concatenate(feas3s, axis=0), jnp.concatenate(feas2s, axis=0),
         jnp.concatenate(feas1s, axis=0), pf6], axis=1)


# speedup vs baseline: 1.0644x; 1.0644x over previous
import functools

import numpy as np

import jax
import jax.numpy as jnp
from jax import lax
from jax.experimental import pallas as pl
from jax.experimental.pallas import tpu as pltpu
from jax.experimental.pallas import tpu_sc as plsc

EPS_BN = 1e-5
BLK = 1000
NC, NS = 2, 16
NW = NC * NS
NBUF = 4


def _fold_pn(params):
    W1, b1, g1, be1, W2, b2, g2, be2 = params
    s = 1.0 / jnp.sqrt(1.0 + EPS_BN)
    A1 = W1.T * (g1 * s)[None, :]
    c1 = (b1 * (g1 * s) + be1).reshape(1, -1)
    A2 = W2.T * (g2 * s)[None, :]
    c2 = (b2 * (g2 * s) + be2).reshape(1, -1)
    return A1, c1, A2, c2


def _kron9(a):
    r, c = a.shape
    eye = jnp.asarray(np.eye(9, dtype=np.float32))
    return (eye[:, None, :, None] * a[None, :, None, :]).reshape(9 * r, 9 * c)


_P54 = np.zeros((216, 54), np.float32)
for _j in range(9):
    for _c in range(6):
        _P54[24 * _j + _c, 6 * _j + _c] = 1.0
_Q54 = np.tile(-np.eye(6, dtype=np.float32), (1, 9))


def _dot(x, y):
    return jnp.dot(x, y, preferred_element_type=jnp.float32)


def _pn2(x, a1, c1, a2, c2):
    return _dot(jnp.maximum(_dot(x, a1) + c1, 0.0), a2) + c2


def _colsumsq_kernel(x_ref, o_ref):
    p = jnp.sum(x_ref[...] * x_ref[...], axis=0, keepdims=True)

    @pl.when(pl.program_id(0) == 0)
    def _():
        o_ref[...] = p

    @pl.when(pl.program_id(0) > 0)
    def _():
        o_ref[...] += p


def _prep_kernel(ss_ref, x_ref, pn_ref, a1_ref, c1_ref, a2_ref, c2_ref,
                 t1_ref, nb_ref, pf6_ref):
    x = x_ref[...]
    inv = 1.0 / jnp.maximum(jnp.sqrt(ss_ref[0:1, 0:3]), 1e-12)
    pf6 = jnp.concatenate([x[:, 0:3] * inv, x[:, 3:6] * (1.0 / 255.0)], axis=1)
    f1 = _pn2(pf6, a1_ref[...], c1_ref[...], a2_ref[...], c2_ref[...])
    sel = jnp.concatenate([x[:, 0:3], x[:, 6:9]], axis=1)
    t1_ref[...] = jnp.concatenate(
        [sel, f1, jnp.zeros((BLK, 6), jnp.float32)], axis=1)
    pf6_ref[...] = pf6
    row = pl.program_id(0) * BLK + lax.broadcasted_iota(jnp.int32, (BLK, 9), 0)
    pnv = pn_ref[...]
    nb_ref[...] = jnp.where(pnv == 0, row, pnv)


def _make_gather(d, ch, b_pad):
    bpw = b_pad // NW
    n_ch = bpw // ch
    assert bpw % ch == 0 and ch % 8 == 0 and n_ch % NBUF == 0 and n_ch >= 2 * NBUF
    mesh = plsc.VectorSubcoreMesh(core_axis_name="c", subcore_axis_name="s")

    @functools.partial(
        pl.kernel,
        mesh=mesh,
        out_type=jax.ShapeDtypeStruct((b_pad, d), jnp.float32),
        scratch_types=[
            pltpu.VMEM((bpw,), jnp.int32),
            pltpu.VMEM((NBUF, ch, d), jnp.float32),
        ] + [pltpu.SemaphoreType.DMA] * (2 * NBUF),
        compiler_params=pltpu.CompilerParams(use_tc_tiling_on_sc=False),
    )
    def gather(table_hbm, idx_hbm, out_hbm, idx_v, bufs, *sems):
        wid = lax.axis_index("s") * NC + lax.axis_index("c")
        base = wid * bpw
        gsems, wsems = sems[:NBUF], sems[NBUF:]
        pltpu.sync_copy(idx_hbm.at[pl.ds(base, bpw)], idx_v)

        def g_copy(g, b):
            return pltpu.make_async_copy(
                table_hbm.at[idx_v.at[pl.ds(g * ch, ch)]], bufs.at[b], gsems[b])

        def w_copy(g, b):
            return pltpu.make_async_copy(
                bufs.at[b], out_hbm.at[pl.ds(base + g * ch, ch)], wsems[b])

        for b in range(NBUF):
            g_copy(b, b).start()

        def body(k, _):
            for b in range(NBUF):
                g = NBUF * k + b
                g_copy(g, b).wait()
                w_copy(g, b).start()
                w_copy(g, b).wait()
                g_copy(g + NBUF, b).start()
            return 0

        lax.fori_loop(0, n_ch // NBUF - 1, body, 0)
        for b in range(NBUF):
            g = n_ch - NBUF + b
            g_copy(g, b).wait()
            w_copy(g, b).start()
            w_copy(g, b).wait()

    return gather


def _s1_kernel(e_ref, t1_ref, a1bd, selb, c1t, a2emb, c2emb,
               u1p, uc1, u2a, uc2, fa1, fc1, fa2, fc2, p54, q54,
               _f1a, _t2a, _xa,
               feas_ref, t2_ref, xyz_ref):
    e = e_ref[...]
    sel6 = t1_ref[:, 0:6]
    h = jnp.maximum(_dot(e, a1bd[...]) + _dot(sel6, selb[...]) + c1t[...], 0.0)
    wemb = _dot(h, a2emb[...]) + c2emb[...]
    m = e * wemb
    feas = _pn2(m, u1p[...], uc1[...], u2a[...], uc2[...])
    feas_ref[...] = feas
    t2_ref[...] = _pn2(feas, fa1[...], fc1[...], fa2[...], fc2[...])
    xyz_ref[...] = _dot(e, p54[...]) + _dot(sel6, q54[...])


def _s2_kernel(e_ref, xyz_ref, a1bd, c1t, a2bd, c2t, u1a, uc1, u2a, uc2,
               fa1, fc1, fa2, fc2, _f2a, _t3a, feas_ref, t3_ref):
    e = e_ref[...]
    h = jnp.maximum(_dot(xyz_ref[...], a1bd[...]) + c1t[...], 0.0)
    w = _dot(h, a2bd[...]) + c2t[...]
    m = e * w
    feas = _pn2(m, u1a[...], uc1[...], u2a[...], uc2[...])
    feas_ref[...] = feas
    t3_ref[...] = _pn2(feas, fa1[...], fc1[...], fa2[...], fc2[...])


def _s3_kernel(e_ref, xyz_ref, a1bd, c1t, a2bd, c2t, u1a, uc1, u2a, uc2,
               f2_ref, f1_ref, pf6_ref, _oa, out_ref):
    e = e_ref[...]
    h = jnp.maximum(_dot(xyz_ref[...], a1bd[...]) + c1t[...], 0.0)
    w = _dot(h, a2bd[...]) + c2t[...]
    m = e * w
    feas = _pn2(m, u1a[...], uc1[...], u2a[...], uc2[...])
    out_ref[...] = jnp.concatenate(
        [feas, f2_ref[...], f1_ref[...], pf6_ref[...]], axis=1)


def _full_spec(shape):
    return pl.BlockSpec(shape, lambda i: (0, 0))


def _row_spec(w):
    return pl.BlockSpec((BLK, w), lambda i: (i, 0))


Q = 4


def kernel(points_features, points_neighbor, p1f, p1w, p1u, p2f, p2w, p2u,
           p3f, p3w, p3u):
    n, m = points_features.shape[0], points_neighbor.shape[1]
    nq = n // Q
    bq = nq * m
    ch24, ch48 = 600, 360
    bq_pad = ((bq + 8 * NW - 1) // (8 * NW)) * (8 * NW)
    while ((bq_pad // NW) % (NBUF * ch24) or (bq_pad // NW) % (NBUF * ch48)
           or bq_pad % 9):
        bq_pad += 8 * NW
    grid = (n // BLK,)
    gridq = (nq // BLK,)
    qb = nq // BLK

    f1p = _fold_pn(p1f)
    w1a1, w1c1, w1a2, w1c2 = _fold_pn(p1w)
    u1a1, u1c1, u1a2, u1c2 = _fold_pn(p1u)
    f2p = _fold_pn(p2f)
    w2a1, w2c1, w2a2, w2c2 = _fold_pn(p2w)
    u2a1, u2c1, u2a2, u2c2 = _fold_pn(p2u)
    f3p = _fold_pn(p3f)
    w3a1, w3c1, w3a2, w3c2 = _fold_pn(p3w)
    u3a1, u3c1, u3a2, u3c2 = _fold_pn(p3u)

    z = jnp.zeros
    s1_a1bd = _kron9(jnp.concatenate([w1a1, z((18, 12), jnp.float32)], axis=0))
    s1_selb = jnp.tile(-w1a1, (1, 9))
    s1_c1t = jnp.tile(w1c1, (1, 9))
    pad6 = z((12, 6), jnp.float32)
    s1_a2emb = _kron9(jnp.concatenate([pad6, w1a2, pad6], axis=1))
    s1_c2emb = jnp.tile(jnp.concatenate(
        [z((1, 6), jnp.float32), w1c2, z((1, 6), jnp.float32)], axis=1), (1, 9))
    s1_u1p = jnp.concatenate([
        z((9, 6, 12), jnp.float32), u1a1.reshape(9, 12, 12),
        z((9, 6, 12), jnp.float32)], axis=1).reshape(216, 12)
    s1_p54 = jnp.asarray(_P54)
    s1_q54 = jnp.asarray(_Q54)

    s2_a1bd = _kron9(w2a1)
    s2_c1t = jnp.tile(w2c1, (1, 9))
    s2_a2bd = _kron9(w2a2)
    s2_c2t = jnp.tile(w2c2, (1, 9))
    s3_a1bd = _kron9(w3a1)
    s3_c1t = jnp.tile(w3c1, (1, 9))
    s3_a2bd = _kron9(w3a2)
    s3_c2t = jnp.tile(w3c2, (1, 9))

    ss = pl.pallas_call(
        _colsumsq_kernel,
        grid=grid,
        in_specs=[_row_spec(9)],
        out_specs=pl.BlockSpec((1, 9), lambda i: (0, 0)),
        out_shape=jax.ShapeDtypeStruct((1, 9), jnp.float32),
    )(points_features)

    t1, nb, pf6 = pl.pallas_call(
        _prep_kernel,
        grid=grid,
        in_specs=[_full_spec((1, 9)), _row_spec(9), _row_spec(9)]
        + [_full_spec(x.shape) for x in f1p],
        out_specs=[_row_spec(24), _row_spec(9), _row_spec(6)],
        out_shape=[jax.ShapeDtypeStruct((n, 24), jnp.float32),
                   jax.ShapeDtypeStruct((n, 9), jnp.int32),
                   jax.ShapeDtypeStruct((n, 6), jnp.float32)],
    )(ss, points_features, points_neighbor, *f1p)

    idxs = [jnp.pad(nb[q * nq:(q + 1) * nq].reshape(-1), (0, bq_pad - bq))
            for q in range(Q)]

    gather24 = _make_gather(24, ch24, bq_pad)
    gather48 = _make_gather(48, ch48, bq_pad)

    def qspec(w, q):
        return pl.BlockSpec((BLK, w), lambda i, _q=q: (i + _q * qb, 0))

    s1_w = (s1_a1bd, s1_selb, s1_c1t, s1_a2emb, s1_c2emb,
            s1_u1p, u1c1, u1a2, u1c2, *f2p, s1_p54, s1_q54)
    s2_w = (s2_a1bd, s2_c1t, s2_a2bd, s2_c2t, u2a1, u2c1, u2a2, u2c2, *f3p)
    s3_w = (s3_a1bd, s3_c1t, s3_a2bd, s3_c2t, u3a1, u3c1, u3a2, u3c2)

    any_spec = pl.BlockSpec(memory_space=pl.ANY)

    feas1 = jnp.zeros((n, 12), jnp.float32)
    t2 = jnp.zeros((n, 24), jnp.float32)
    xyz = jnp.zeros((n, 54), jnp.float32)
    for q in range(Q):
        e1 = gather24(t1, idxs[q]).reshape(bq_pad // 9, 9 * 24)
        feas1, t2, xyz = pl.pallas_call(
            _s1_kernel,
            grid=gridq,
            in_specs=[_row_spec(216), qspec(24, q)]
            + [_full_spec(x.shape) for x in s1_w] + [any_spec] * 3,
            out_specs=[qspec(12, q), qspec(24, q), qspec(54, q)],
            out_shape=[jax.ShapeDtypeStruct((n, 12), jnp.float32),
                       jax.ShapeDtypeStruct((n, 24), jnp.float32),
                       jax.ShapeDtypeStruct((n, 54), jnp.float32)],
            input_output_aliases={2 + len(s1_w): 0, 3 + len(s1_w): 1,
                                  4 + len(s1_w): 2},
        )(e1, t1, *s1_w, feas1, t2, xyz)

    feas2 = jnp.zeros((n, 24), jnp.float32)
    t3 = jnp.zeros((n, 48), jnp.float32)
    for q in range(Q):
        e2 = gather24(t2, idxs[q]).reshape(bq_pad // 9, 9 * 24)
        feas2, t3 = pl.pallas_call(
            _s2_kernel,
            grid=gridq,
            in_specs=[_row_spec(216), qspec(54, q)]
            + [_full_spec(x.shape) for x in s2_w] + [any_spec] * 2,
            out_specs=[qspec(24, q), qspec(48, q)],
            out_shape=[jax.ShapeDtypeStruct((n, 24), jnp.float32),
                       jax.ShapeDtypeStruct((n, 48), jnp.float32)],
            input_output_aliases={2 + len(s2_w): 0, 3 + len(s2_w): 1},
        )(e2, xyz, *s2_w, feas2, t3)

    out = jnp.zeros((n, 90), jnp.float32)
    for q in range(Q):
        e3 = gather48(t3, idxs[q]).reshape(bq_pad // 9, 9 * 48)
        out = pl.pallas_call(
            _s3_kernel,
            grid=gridq,
            in_specs=[_row_spec(432), qspec(54, q)]
            + [_full_spec(x.shape) for x in s3_w]
            + [qspec(24, q), qspec(12, q), qspec(6, q), any_spec],
            out_specs=qspec(90, q),
            out_shape=jax.ShapeDtypeStruct((n, 90), jnp.float32),
            input_output_aliases={5 + len(s3_w): 0},
        )(e3, xyz, *s3_w, feas2, feas1, pf6, out)

    return out
